# baseline (speedup 1.0000x reference)
import functools
import math

import jax
import jax.numpy as jnp
from jax import lax
from jax.experimental import pallas as pl
from jax.experimental.pallas import tpu as pltpu
from jax.experimental.pallas import tpu_sc as plsc

N_NODES = 10000
N_EDGES = 80000
D = 1024
NUM_GRAPHS = 64
EPS = 1e-5

_TM = 1000


def _mm_body(x_ref, w_ref, o_ref):
    o_ref[...] = lax.dot_general(
        x_ref[...], w_ref[...], (((1,), (0,)), ((), ())),
        preferred_element_type=jnp.float32,
        precision=lax.Precision.HIGHEST)


def _k1_matmul(x, wcat):
    grid = (N_NODES // _TM, 4)
    return pl.pallas_call(
        _mm_body,
        grid=grid,
        in_specs=[
            pl.BlockSpec((_TM, D), lambda m, n: (m, 0)),
            pl.BlockSpec((D, D), lambda m, n: (0, n)),
        ],
        out_specs=pl.BlockSpec((_TM, D), lambda m, n: (m, n)),
        out_shape=jax.ShapeDtypeStruct((N_NODES, 4 * D), jnp.float32),
    )(x, wcat)


_SL = 16
_NSL = D // _SL
_HALF = N_NODES // 2
_SUBROW = 128
_SUB = 5
_CHUNK = _SUB * _SUBROW
_E_PAD = 80640
_NCHUNK = _E_PAD // _CHUNK
_GROUPS = _CHUNK // 16
_UNITS = 8
_NEG = float("-inf")
_YROWS = N_NODES * 4 * D // _SL


def _sc_body(src_hbm, dst_hbm, yf_hbm, m_hbm, m_v, sv, dv, iv, gv,
             sem_s0, sem_s1, sem_d0, sem_d1, sem_g0, sem_g1):
    nc = 2
    wid = lax.axis_index("s") * nc + lax.axis_index("c")
    sem_s = (sem_s0, sem_s1)
    sem_d = (sem_d0, sem_d1)
    sem_g = (sem_g0, sem_g1)
    iota16 = lax.iota(jnp.int32, 16)

    def src_cp(n, b):
        return pltpu.make_async_copy(
            src_hbm.at[pl.ds(n * _CHUNK, _CHUNK)], sv.at[b], sem_s[b])

    def dst_cp(n, b):
        return pltpu.make_async_copy(
            dst_hbm.at[pl.ds(n * _CHUNK, _CHUNK)], dv.at[b], sem_d[b])

    def gat_cps(b):
        return [pltpu.make_async_copy(
            yf_hbm.at[iv.at[b * _SUB + j]],
            gv.at[b].at[pl.ds(j * _SUBROW, _SUBROW)],
            sem_g[b]) for j in range(_SUB)]

    def compute_idx(row_off, b):
        for j in range(_SUB):
            for g in range(_SUBROW // 16):
                srcv = sv[b, pl.ds(j * _SUBROW + g * 16, 16)]
                iv[b * _SUB + j, pl.ds(g * 16, 16)] = srcv * 256 + row_off

    i16 = lax.iota(jnp.int32, 16)
    p01 = i16 // 8
    col8 = i16 & 7
    swapidx = i16 ^ 8
    pairswap = i16 ^ 1
    negfull = jnp.full((16,), _NEG, jnp.float32)

    def take(v, idx):
        return lax.gather(
            v, idx[:, None],
            lax.GatherDimensionNumbers(
                offset_dims=(), collapsed_slice_dims=(0,),
                start_index_map=(0,)),
            slice_sizes=(1,),
            mode=lax.GatherScatterMode.PROMISE_IN_BOUNDS)

    def unit_body(u, _):
        ug = wid * _UNITS + u
        blk = ug // 128
        rem = ug % 128
        sl16 = rem // 2
        coloff = (rem % 2) * 8
        row_off = 128 + blk * _NSL + sl16

        def init_row(i, _):
            plsc.store_scatter(
                m_v, [jnp.full((16,), 2 * i, jnp.int32) + p01, col8],
                negfull)
            return _
        lax.fori_loop(0, (N_NODES + 2) // 2, init_row, None)

        src_cp(0, 0).start()
        src_cp(0, 0).wait()
        compute_idx(row_off, 0)
        for cp in gat_cps(0):
            cp.start()
        dst_cp(0, 0).start()
        src_cp(1, 1).start()

        colidx = jnp.full((16,), coloff, jnp.int32) + col8

        def compute_chunk(b):
            def group_body(g, _):
                e0 = g * 16
                dstv = dv[b, pl.ds(e0, 16)]
                erow_b = jnp.full((16,), e0, jnp.int32) + p01
                hasdup = jnp.any(dstv == take(dstv, pairswap))

                @pl.when(jnp.logical_not(hasdup))
                def _fast():
                    for p in range(8):
                        erow = erow_b + (2 * p)
                        dgath = take(dstv, p01 + (2 * p))
                        grow = plsc.load_gather(gv.at[b], [erow, colidx])
                        cur = plsc.load_gather(m_v, [dgath, col8])
                        plsc.store_scatter(
                            m_v, [dgath, col8], jnp.maximum(cur, grow))

                @pl.when(hasdup)
                def _slow():
                    for p in range(8):
                        erow = erow_b + (2 * p)
                        dgath = take(dstv, p01 + (2 * p))
                        dswap = take(dgath, swapidx)
                        eq = dgath == dswap
                        grow = plsc.load_gather(gv.at[b], [erow, colidx])
                        gswap = take(grow, swapidx)
                        gsel = jnp.where(eq, jnp.maximum(grow, gswap), grow)
                        cur = plsc.load_gather(m_v, [dgath, col8])
                        plsc.store_scatter(
                            m_v, [dgath, col8], jnp.maximum(cur, gsel))
                return _
            lax.fori_loop(0, _GROUPS, group_body, None)

        def chunk_pair(cc, _):
            for b in (0, 1):
                n = cc * 2 + b
                for cp in gat_cps(b):
                    cp.wait()
                dst_cp(n, b).wait()
                compute_chunk(b)
                nb = 1 - b

                @pl.when(n + 1 < _NCHUNK)
                def _issue_next():
                    src_cp(n + 1, nb).wait()
                    compute_idx(row_off, nb)
                    for cp in gat_cps(nb):
                        cp.start()
                    dst_cp(n + 1, nb).start()

                @pl.when(n + 2 < _NCHUNK)
                def _issue_next2():
                    src_cp(n + 2, b).start()
            return _

        lax.fori_loop(0, _NCHUNK // 2, chunk_pair, None)

        pltpu.sync_copy(
            m_v.at[pl.ds(0, N_NODES), :],
            m_hbm.at[blk].at[pl.ds(0, N_NODES), pl.ds(rem * 8, 8)])
        return _

    lax.fori_loop(0, _UNITS, unit_body, None)


def _k3_scatter_max(src, dst, yf):
    mesh = plsc.VectorSubcoreMesh(core_axis_name="c", subcore_axis_name="s")
    f = pl.kernel(
        _sc_body,
        out_type=jax.ShapeDtypeStruct((2, N_NODES, D), jnp.float32),
        mesh=mesh,
        compiler_params=pltpu.CompilerParams(
            use_tc_tiling_on_sc=False, needs_layout_passes=False),
        scratch_types=[
            pltpu.VMEM((N_NODES + 2, 8), jnp.float32),
            pltpu.VMEM((2, _CHUNK), jnp.int32),
            pltpu.VMEM((2, _CHUNK), jnp.int32),
            pltpu.VMEM((2 * _SUB, _SUBROW), jnp.int32),
            pltpu.VMEM((2, _CHUNK, _SL), jnp.float32),
            pltpu.SemaphoreType.DMA,
            pltpu.SemaphoreType.DMA,
            pltpu.SemaphoreType.DMA,
            pltpu.SemaphoreType.DMA,
            pltpu.SemaphoreType.DMA,
            pltpu.SemaphoreType.DMA,
        ],
    )
    return f(src, dst, yf)


_NSTEP = N_NODES // _TM


def _k4_body(x_ref, a0_ref, a1_ref, m0_ref, m1_ref, b_ref, p_ref,
             iw_ref, bw_ref, row_ref, o_ref, zx, z0, z1, zc):
    m = pl.program_id(0)

    @pl.when(m == 0)
    def _init():
        zx[...] = jnp.zeros_like(zx)
        z0[...] = jnp.zeros_like(z0)
        z1[...] = jnp.zeros_like(z1)
        zc[...] = jnp.zeros_like(zc)

    bt = b_ref[...]
    iota_g = lax.broadcasted_iota(jnp.int32, (1, NUM_GRAPHS), 1).astype(
        jnp.float32)
    oh = (bt == iota_g).astype(jnp.float32)

    def otm(h):
        return lax.dot_general(oh, h, (((0,), (0,)), ((), ())),
                               preferred_element_type=jnp.float32,
                               precision=lax.Precision.HIGHEST)

    zx[...] += otm(x_ref[...])
    m0 = m0_ref[0]
    fin0 = m0[:, 0:1] != _NEG
    h0 = jnp.where(fin0, a0_ref[...] + p_ref[0:1, :] + m0, 0.0)
    z0[...] += otm(h0)
    m1 = m1_ref[0]
    fin1 = m1[:, 0:1] != _NEG
    h1 = jnp.where(fin1, a1_ref[...] + p_ref[1:2, :] + m1, 0.0)
    z1[...] += otm(h1)
    zc[...] += otm(jnp.ones((_TM, 128), jnp.float32))

    @pl.when(m == _NSTEP - 1)
    def _final():
        cnt = jnp.maximum(zc[:, 0:1], 1.0)
        invc = 1.0 / cnt

        def dot(a, b):
            return lax.dot_general(a, b, (((1,), (0,)), ((), ())),
                                   preferred_element_type=jnp.float32,
                                   precision=lax.Precision.HIGHEST)

        xbar = zx[...] * invc
        acc = jnp.zeros((NUM_GRAPHS, 128), jnp.float32)
        for i in range(2):
            hb = dot(xbar, iw_ref[i]) + p_ref[6 + i:7 + i, :]
            g1 = hb * p_ref[2 + i:3 + i, :] + p_ref[4 + i:5 + i, :]
            pp = dot(g1, bw_ref[i]) + p_ref[16 + i:17 + i, 0:128]
            acc += pp * p_ref[8 + i:9 + i, 0:128] + p_ref[12 + i:13 + i, 0:128]
        for j in range(2):
            k = 2 + j
            gj = (z0[...] if j == 0 else z1[...]) * invc
            pp = dot(gj, bw_ref[k]) + p_ref[16 + k:17 + k, 0:128]
            acc += pp * p_ref[8 + k:9 + k, 0:128] + p_ref[12 + k:13 + k, 0:128]
        o_ref[...] = dot(acc, row_ref[...]) + p_ref[20:21, 0:128]


def _k4_pool(x, y, mout, batch_f, p, iw, bwp, rowp):
    grid = (_NSTEP,)
    return pl.pallas_call(
        _k4_body,
        grid=grid,
        in_specs=[
            pl.BlockSpec((_TM, D), lambda m: (m, 0)),
            pl.BlockSpec((_TM, D), lambda m: (m, 0)),
            pl.BlockSpec((_TM, D), lambda m: (m, 1)),
            pl.BlockSpec((1, _TM, D), lambda m: (0, m, 0)),
            pl.BlockSpec((1, _TM, D), lambda m: (1, m, 0)),
            pl.BlockSpec((_TM, 1), lambda m: (m, 0)),
            pl.BlockSpec((24, D), lambda m: (0, 0)),
            pl.BlockSpec((2, D, D), lambda m: (0, 0, 0)),
            pl.BlockSpec((4, D, 128), lambda m: (0, 0, 0)),
            pl.BlockSpec((128, 128), lambda m: (0, 0)),
        ],
        out_specs=pl.BlockSpec((NUM_GRAPHS, 128), lambda m: (0, 0)),
        out_shape=jax.ShapeDtypeStruct((NUM_GRAPHS, 128), jnp.float32),
        scratch_shapes=[
            pltpu.VMEM((NUM_GRAPHS, D), jnp.float32),
            pltpu.VMEM((NUM_GRAPHS, D), jnp.float32),
            pltpu.VMEM((NUM_GRAPHS, D), jnp.float32),
            pltpu.VMEM((NUM_GRAPHS, 128), jnp.float32),
        ],
    )(x, y, y, mout, mout, batch_f, p, iw, bwp, rowp)


def kernel(x, edge_index, batch, init_W, init_b, init_g, init_bt,
           edge_W, edge_b, edge_g, edge_bt, blk_W, blk_b, blk_g, blk_bt,
           ro_W, ro_b):
    inv = 1.0 / math.sqrt(1.0 + EPS)
    f32 = jnp.float32

    s0 = edge_g[0] * inv
    s1 = edge_g[1] * inv
    u0 = (edge_W[0][:D] - edge_W[0][D:]) * s0[None, :]
    u1 = (edge_W[1][:D] - edge_W[1][D:]) * s1[None, :]
    v0 = edge_W[0][D:] * s0[None, :]
    v1 = edge_W[1][D:] * s1[None, :]
    wcat = jnp.concatenate([u0, u1, v0, v1], axis=1)

    c0 = edge_b[0] * s0 + edge_bt[0]
    c1 = edge_b[1] * s1 + edge_bt[1]
    p = jnp.zeros((24, D), f32)
    p = p.at[0].set(c0).at[1].set(c1)
    p = p.at[2].set(init_g[0] * inv).at[3].set(init_g[1] * inv)
    p = p.at[4].set(init_bt[0]).at[5].set(init_bt[1])
    p = p.at[6].set(init_b[0]).at[7].set(init_b[1])
    p = p.at[8:12, 0:2].set(blk_g * inv)
    p = p.at[12:16, 0:2].set(blk_bt)
    p = p.at[16:20, 0:2].set(blk_b)
    p = p.at[20, 0:2].set(ro_b)
    bwp = jnp.zeros((4, D, 128), f32).at[:, :, 0:2].set(blk_W)
    rowp = jnp.zeros((128, 128), f32).at[0:2, 0:2].set(ro_W)
    batch_f = batch.astype(f32).reshape(N_NODES, 1)
    pad = _E_PAD - N_EDGES
    src = jnp.concatenate([edge_index[0], jnp.zeros((pad,), jnp.int32)])
    dst = jnp.concatenate(
        [edge_index[1], jnp.full((pad,), N_NODES, jnp.int32)])

    y = _k1_matmul(x, wcat)
    yf = y.reshape(_YROWS, _SL)
    mout = _k3_scatter_max(src, dst, yf)
    out = _k4_pool(x, y, mout, batch_f, p, init_W, bwp, rowp)
    return out[:, 0:2]


# speedup vs baseline: 1.1177x; 1.1177x over previous
import functools
import math

import jax
import jax.numpy as jnp
from jax import lax
from jax.experimental import pallas as pl
from jax.experimental.pallas import tpu as pltpu
from jax.experimental.pallas import tpu_sc as plsc

N_NODES = 10000
N_EDGES = 80000
D = 1024
NUM_GRAPHS = 64
EPS = 1e-5

_TM = 1000


def _mm_body(x_ref, w_ref, o_ref):
    o_ref[...] = lax.dot_general(
        x_ref[...], w_ref[...], (((1,), (0,)), ((), ())),
        preferred_element_type=jnp.float32,
        precision=lax.Precision.HIGHEST)


def _k1_matmul(x, w2):
    grid = (N_NODES // _TM, 2)
    return pl.pallas_call(
        _mm_body,
        grid=grid,
        in_specs=[
            pl.BlockSpec((_TM, D), lambda m, n: (m, 0)),
            pl.BlockSpec((D, D), lambda m, n: (0, n)),
        ],
        out_specs=pl.BlockSpec((_TM, D), lambda m, n: (m, n)),
        out_shape=jax.ShapeDtypeStruct((N_NODES, 2 * D), jnp.float32),
    )(x, w2)


_SL = 16
_NSL = D // _SL
_HALF = N_NODES // 2
_SUBROW = 128
_SUB = 5
_CHUNK = _SUB * _SUBROW
_E_PAD = 80640
_NCHUNK = _E_PAD // _CHUNK
_GROUPS = _CHUNK // 16
_UNITS = 8
_NEG = float("-inf")
_YROWS = N_NODES * 2 * D // _SL


def _sc_body(src_hbm, dst_hbm, yf_hbm, m_hbm, m_v, sv, dv, iv, gv,
             sem_s0, sem_s1, sem_d0, sem_d1, sem_g0, sem_g1):
    nc = 2
    wid = lax.axis_index("s") * nc + lax.axis_index("c")
    sem_s = (sem_s0, sem_s1)
    sem_d = (sem_d0, sem_d1)
    sem_g = (sem_g0, sem_g1)
    iota16 = lax.iota(jnp.int32, 16)

    def src_cp(n, b):
        return pltpu.make_async_copy(
            src_hbm.at[pl.ds(n * _CHUNK, _CHUNK)], sv.at[b], sem_s[b])

    def dst_cp(n, b):
        return pltpu.make_async_copy(
            dst_hbm.at[pl.ds(n * _CHUNK, _CHUNK)], dv.at[b], sem_d[b])

    def gat_cps(b):
        return [pltpu.make_async_copy(
            yf_hbm.at[iv.at[b * _SUB + j]],
            gv.at[b].at[pl.ds(j * _SUBROW, _SUBROW)],
            sem_g[b]) for j in range(_SUB)]

    def compute_idx(row_off, b):
        for j in range(_SUB):
            for g in range(_SUBROW // 16):
                srcv = sv[b, pl.ds(j * _SUBROW + g * 16, 16)]
                iv[b * _SUB + j, pl.ds(g * 16, 16)] = srcv * 128 + row_off

    i16 = lax.iota(jnp.int32, 16)
    p01 = i16 // 8
    col8 = i16 & 7
    swapidx = i16 ^ 8
    pairswap = i16 ^ 1
    negfull = jnp.full((16,), _NEG, jnp.float32)

    def take(v, idx):
        return lax.gather(
            v, idx[:, None],
            lax.GatherDimensionNumbers(
                offset_dims=(), collapsed_slice_dims=(0,),
                start_index_map=(0,)),
            slice_sizes=(1,),
            mode=lax.GatherScatterMode.PROMISE_IN_BOUNDS)

    def unit_body(u, _):
        ug = wid * _UNITS + u
        blk = ug // 128
        rem = ug % 128
        sl16 = rem // 2
        coloff = (rem % 2) * 8
        row_off = blk * _NSL + sl16

        def init_row(i, _):
            plsc.store_scatter(
                m_v, [jnp.full((16,), 2 * i, jnp.int32) + p01, col8],
                negfull)
            return _
        lax.fori_loop(0, (N_NODES + 2) // 2, init_row, None)

        src_cp(0, 0).start()
        src_cp(0, 0).wait()
        compute_idx(row_off, 0)
        for cp in gat_cps(0):
            cp.start()
        dst_cp(0, 0).start()
        src_cp(1, 1).start()

        colidx = jnp.full((16,), coloff, jnp.int32) + col8

        def compute_chunk(b):
            def group_body(g, _):
                e0 = g * 16
                dstv = dv[b, pl.ds(e0, 16)]
                erow_b = jnp.full((16,), e0, jnp.int32) + p01
                for p in range(8):
                    erow = erow_b + (2 * p)
                    dgath = take(dstv, p01 + (2 * p))
                    dswap = take(dgath, swapidx)
                    eq = dgath == dswap
                    grow = plsc.load_gather(gv.at[b], [erow, colidx])
                    gswap = take(grow, swapidx)
                    gsel = jnp.where(eq, jnp.maximum(grow, gswap), grow)
                    cur = plsc.load_gather(m_v, [dgath, col8])
                    plsc.store_scatter(
                        m_v, [dgath, col8], jnp.maximum(cur, gsel))
                return _
            lax.fori_loop(0, _GROUPS, group_body, None)

        def chunk_pair(cc, _):
            for b in (0, 1):
                n = cc * 2 + b
                for cp in gat_cps(b):
                    cp.wait()
                dst_cp(n, b).wait()
                compute_chunk(b)
                nb = 1 - b

                @pl.when(n + 1 < _NCHUNK)
                def _issue_next():
                    src_cp(n + 1, nb).wait()
                    compute_idx(row_off, nb)
                    for cp in gat_cps(nb):
                        cp.start()
                    dst_cp(n + 1, nb).start()

                @pl.when(n + 2 < _NCHUNK)
                def _issue_next2():
                    src_cp(n + 2, b).start()
            return _

        lax.fori_loop(0, _NCHUNK // 2, chunk_pair, None)

        pltpu.sync_copy(
            m_v.at[pl.ds(0, N_NODES), :],
            m_hbm.at[blk].at[pl.ds(0, N_NODES), pl.ds(rem * 8, 8)])
        return _

    lax.fori_loop(0, _UNITS, unit_body, None)


def _k3_scatter_max(src, dst, yf):
    mesh = plsc.VectorSubcoreMesh(core_axis_name="c", subcore_axis_name="s")
    f = pl.kernel(
        _sc_body,
        out_type=jax.ShapeDtypeStruct((2, N_NODES, D), jnp.float32),
        mesh=mesh,
        compiler_params=pltpu.CompilerParams(
            use_tc_tiling_on_sc=False, needs_layout_passes=False),
        scratch_types=[
            pltpu.VMEM((N_NODES + 2, 8), jnp.float32),
            pltpu.VMEM((2, _CHUNK), jnp.int32),
            pltpu.VMEM((2, _CHUNK), jnp.int32),
            pltpu.VMEM((2 * _SUB, _SUBROW), jnp.int32),
            pltpu.VMEM((2, _CHUNK, _SL), jnp.float32),
            pltpu.SemaphoreType.DMA,
            pltpu.SemaphoreType.DMA,
            pltpu.SemaphoreType.DMA,
            pltpu.SemaphoreType.DMA,
            pltpu.SemaphoreType.DMA,
            pltpu.SemaphoreType.DMA,
        ],
    )
    return f(src, dst, yf)


_NSTEP = N_NODES // _TM


def _k4_body(x_ref, a0_ref, a1_ref, m0_ref, m1_ref, b_ref, p_ref,
             iw_ref, bw_ref, row_ref, o_ref, zx, z0, z1, zc):
    m = pl.program_id(0)

    @pl.when(m == 0)
    def _init():
        zx[...] = jnp.zeros_like(zx)
        z0[...] = jnp.zeros_like(z0)
        z1[...] = jnp.zeros_like(z1)
        zc[...] = jnp.zeros_like(zc)

    bt = b_ref[...]
    iota_g = lax.broadcasted_iota(jnp.int32, (1, NUM_GRAPHS), 1).astype(
        jnp.float32)
    oh = (bt == iota_g).astype(jnp.float32)

    def otm(h):
        return lax.dot_general(oh, h, (((0,), (0,)), ((), ())),
                               preferred_element_type=jnp.float32,
                               precision=lax.Precision.HIGHEST)

    zx[...] += otm(x_ref[...])
    m0 = m0_ref[0]
    fin0 = m0[:, 0:1] != _NEG
    h0 = jnp.where(fin0, a0_ref[...] + p_ref[0:1, :] + m0, 0.0)
    z0[...] += otm(h0)
    m1 = m1_ref[0]
    fin1 = m1[:, 0:1] != _NEG
    h1 = jnp.where(fin1, a1_ref[...] + p_ref[1:2, :] + m1, 0.0)
    z1[...] += otm(h1)
    zc[...] += otm(jnp.ones((_TM, 128), jnp.float32))

    @pl.when(m == _NSTEP - 1)
    def _final():
        cnt = jnp.maximum(zc[:, 0:1], 1.0)
        invc = 1.0 / cnt

        def dot(a, b):
            return lax.dot_general(a, b, (((1,), (0,)), ((), ())),
                                   preferred_element_type=jnp.float32,
                                   precision=lax.Precision.HIGHEST)

        xbar = zx[...] * invc
        acc = jnp.zeros((NUM_GRAPHS, 128), jnp.float32)
        for i in range(2):
            hb = dot(xbar, iw_ref[i]) + p_ref[6 + i:7 + i, :]
            g1 = hb * p_ref[2 + i:3 + i, :] + p_ref[4 + i:5 + i, :]
            pp = dot(g1, bw_ref[i]) + p_ref[16 + i:17 + i, 0:128]
            acc += pp * p_ref[8 + i:9 + i, 0:128] + p_ref[12 + i:13 + i, 0:128]
        for j in range(2):
            k = 2 + j
            gj = (z0[...] if j == 0 else z1[...]) * invc
            pp = dot(gj, bw_ref[k]) + p_ref[16 + k:17 + k, 0:128]
            acc += pp * p_ref[8 + k:9 + k, 0:128] + p_ref[12 + k:13 + k, 0:128]
        o_ref[...] = dot(acc, row_ref[...]) + p_ref[20:21, 0:128]


def _k4_pool(x, y, mout, batch_f, p, iw, bwp, rowp):
    grid = (_NSTEP,)
    return pl.pallas_call(
        _k4_body,
        grid=grid,
        in_specs=[
            pl.BlockSpec((_TM, D), lambda m: (m, 0)),
            pl.BlockSpec((_TM, D), lambda m: (m, 0)),
            pl.BlockSpec((_TM, D), lambda m: (m, 1)),
            pl.BlockSpec((1, _TM, D), lambda m: (0, m, 0)),
            pl.BlockSpec((1, _TM, D), lambda m: (1, m, 0)),
            pl.BlockSpec((_TM, 1), lambda m: (m, 0)),
            pl.BlockSpec((24, D), lambda m: (0, 0)),
            pl.BlockSpec((2, D, D), lambda m: (0, 0, 0)),
            pl.BlockSpec((4, D, 128), lambda m: (0, 0, 0)),
            pl.BlockSpec((128, 128), lambda m: (0, 0)),
        ],
        out_specs=pl.BlockSpec((NUM_GRAPHS, 128), lambda m: (0, 0)),
        out_shape=jax.ShapeDtypeStruct((NUM_GRAPHS, 128), jnp.float32),
        scratch_shapes=[
            pltpu.VMEM((NUM_GRAPHS, D), jnp.float32),
            pltpu.VMEM((NUM_GRAPHS, D), jnp.float32),
            pltpu.VMEM((NUM_GRAPHS, D), jnp.float32),
            pltpu.VMEM((NUM_GRAPHS, 128), jnp.float32),
        ],
    )(x, y, y, mout, mout, batch_f, p, iw, bwp, rowp)


def kernel(x, edge_index, batch, init_W, init_b, init_g, init_bt,
           edge_W, edge_b, edge_g, edge_bt, blk_W, blk_b, blk_g, blk_bt,
           ro_W, ro_b):
    inv = 1.0 / math.sqrt(1.0 + EPS)
    f32 = jnp.float32

    s0 = edge_g[0] * inv
    s1 = edge_g[1] * inv
    u0 = (edge_W[0][:D] - edge_W[0][D:]) * s0[None, :]
    u1 = (edge_W[1][:D] - edge_W[1][D:]) * s1[None, :]
    v0 = edge_W[0][D:] * s0[None, :]
    v1 = edge_W[1][D:] * s1[None, :]
    wa = jnp.concatenate([u0, u1], axis=1)
    wb = jnp.concatenate([v0, v1], axis=1)

    c0 = edge_b[0] * s0 + edge_bt[0]
    c1 = edge_b[1] * s1 + edge_bt[1]
    p = jnp.zeros((24, D), f32)
    p = p.at[0].set(c0).at[1].set(c1)
    p = p.at[2].set(init_g[0] * inv).at[3].set(init_g[1] * inv)
    p = p.at[4].set(init_bt[0]).at[5].set(init_bt[1])
    p = p.at[6].set(init_b[0]).at[7].set(init_b[1])
    p = p.at[8:12, 0:2].set(blk_g * inv)
    p = p.at[12:16, 0:2].set(blk_bt)
    p = p.at[16:20, 0:2].set(blk_b)
    p = p.at[20, 0:2].set(ro_b)
    bwp = jnp.zeros((4, D, 128), f32).at[:, :, 0:2].set(blk_W)
    rowp = jnp.zeros((128, 128), f32).at[0:2, 0:2].set(ro_W)
    batch_f = batch.astype(f32).reshape(N_NODES, 1)
    pad = _E_PAD - N_EDGES
    src = jnp.concatenate([edge_index[0], jnp.zeros((pad,), jnp.int32)])
    dst = jnp.concatenate(
        [edge_index[1], jnp.full((pad,), N_NODES, jnp.int32)])

    yb = _k1_matmul(x, wb)
    mout = _k3_scatter_max(src, dst, yb.reshape(_YROWS, _SL))
    ya = _k1_matmul(x, wa)
    out = _k4_pool(x, ya, mout, batch_f, p, init_W, bwp, rowp)
    return out[:, 0:2]
